# baseline (speedup 1.0000x reference)
import functools

import jax
import jax.numpy as jnp
from jax import lax
from jax.experimental import pallas as pl
from jax.experimental.pallas import tpu as pltpu
from jax.experimental.pallas import tpu_sc as plsc

_NC = 2
_NS = 16
_NW = _NC * _NS
_LANES = 16

_N_LAST = 10
_BLK = 128


def _make_sc_call(B, D, n_blocks, s_per_w):
    nq = D // _LANES

    def body(idx_hbm, li_hbm, ui_hbm, item_out, user_out,
             idx_v, bufs, acc, ub, ui_v, *sems):
        c = lax.axis_index("c")
        s = lax.axis_index("s")
        wid = s * _NC + c

        pltpu.sync_copy(idx_hbm.at[pl.ds(wid * s_per_w, s_per_w)], idx_v)
        pltpu.sync_copy(ui_hbm, ui_v)

        def fill_row(r, carry):
            for q in range(nq):
                sl = pl.ds(q * _LANES, _LANES)
                ub[r, sl] = ui_v[0, sl]
            return carry
        lax.fori_loop(0, _BLK, fill_row, 0, unroll=4)

        for t in range(s_per_w):
            row0 = (wid * s_per_w + t) * _BLK
            copies = [
                pltpu.async_copy(li_hbm.at[idx_v.at[t, j]], bufs.at[j], sems[j])
                for j in range(_N_LAST)
            ]
            for j in range(_N_LAST):
                copies[j].wait()

                def accum(r, carry, j=j):
                    for q in range(nq):
                        sl = pl.ds(q * _LANES, _LANES)
                        v = bufs[j, r, sl]
                        if j == 0:
                            acc[r, sl] = v
                        else:
                            acc[r, sl] += v
                    return carry
                lax.fori_loop(0, _BLK, accum, 0, unroll=8)

            pltpu.sync_copy(acc, item_out.at[pl.ds(row0, _BLK)])
            pltpu.sync_copy(ub, user_out.at[pl.ds(row0, _BLK)])

    return pl.kernel(
        body,
        out_type=(
            jax.ShapeDtypeStruct((B, D), jnp.float32),
            jax.ShapeDtypeStruct((B, D), jnp.float32),
        ),
        mesh=plsc.VectorSubcoreMesh(
            core_axis_name="c", subcore_axis_name="s",
            num_cores=_NC, num_subcores=_NS,
        ),
        scratch_types=(
            [
                pltpu.VMEM((s_per_w, _N_LAST, _BLK), jnp.int32),
                pltpu.VMEM((_N_LAST, _BLK, D), jnp.float32),
                pltpu.VMEM((_BLK, D), jnp.float32),
                pltpu.VMEM((_BLK, D), jnp.float32),
                pltpu.VMEM((1, D), jnp.float32),
            ]
            + [pltpu.SemaphoreType.DMA] * _N_LAST
        ),
        compiler_params=pltpu.CompilerParams(use_tc_tiling_on_sc=False),
    )


def kernel(input, LI_emb, UI_emb, n_iter):
    B, L = input.shape
    V, D = LI_emb.shape

    last = lax.dynamic_slice_in_dim(input, L - n_iter, _N_LAST, axis=1)

    n_blocks = B // _BLK
    idx_arr = last.T.reshape(_N_LAST, n_blocks, _BLK).transpose(1, 0, 2)

    s_per_w = n_blocks // _NW
    item, user = _make_sc_call(B, D, n_blocks, s_per_w)(idx_arr, LI_emb, UI_emb)
    return item, user


# speedup vs baseline: 1.3904x; 1.3904x over previous
import functools

import jax
import jax.numpy as jnp
from jax import lax
from jax.experimental import pallas as pl
from jax.experimental.pallas import tpu as pltpu
from jax.experimental.pallas import tpu_sc as plsc

_NC = 2
_NS = 16
_NW = _NC * _NS
_LANES = 16

_N_LAST = 10
_BLK = 128

_TCHUNK = 7936


def _make_tc_pack(V, D):
    grid = -(-V // _TCHUNK)

    def tbody(in_ref, out_ref):
        t = in_ref[...].T
        t3 = t.reshape(_TCHUNK // 2, 2, D)
        out_ref[...] = jnp.concatenate([t3[:, 0, :], t3[:, 1, :]], axis=1)

    return pl.pallas_call(
        tbody,
        grid=(grid,),
        in_specs=[pl.BlockSpec((D, _TCHUNK), lambda i: (0, i))],
        out_specs=pl.BlockSpec((_TCHUNK // 2, 2 * D), lambda i: (i, 0)),
        out_shape=jax.ShapeDtypeStruct((V // 2, 2 * D), jnp.float32),
    )


def _make_sc_call(B, D, n_blocks, s_per_w):
    nq = D // _LANES

    def body(idx_hbm, li_hbm, ui_hbm, item_out, user_out,
             idx_v, bufs, acc, ub, ui_v, *sems):
        c = lax.axis_index("c")
        s = lax.axis_index("s")
        wid = s * _NC + c

        pltpu.sync_copy(idx_hbm.at[pl.ds(wid * s_per_w, s_per_w)], idx_v)
        pltpu.sync_copy(ui_hbm, ui_v)

        def fill_row(r, carry):
            for q in range(nq):
                sl = pl.ds(q * _LANES, _LANES)
                ub[r, sl] = ui_v[0, sl]
            return carry
        lax.fori_loop(0, _BLK, fill_row, 0, unroll=4)

        for t in range(s_per_w):
            row0 = (wid * s_per_w + t) * _BLK
            copies = [
                pltpu.async_copy(li_hbm.at[idx_v.at[t, j]], bufs.at[j], sems[j])
                for j in range(_N_LAST)
            ]
            for j in range(_N_LAST):
                copies[j].wait()

                def accum(r, carry, j=j):
                    for q in range(nq):
                        sl = pl.ds(q * _LANES, _LANES)
                        v = bufs[j, r, sl]
                        if j == 0:
                            acc[r, sl] = v
                        else:
                            acc[r, sl] += v
                    return carry
                lax.fori_loop(0, _BLK, accum, 0, unroll=8)

            pltpu.sync_copy(acc, item_out.at[pl.ds(row0, _BLK)])
            pltpu.sync_copy(ub, user_out.at[pl.ds(row0, _BLK)])

    return pl.kernel(
        body,
        out_type=(
            jax.ShapeDtypeStruct((B, D), jnp.float32),
            jax.ShapeDtypeStruct((B, D), jnp.float32),
        ),
        mesh=plsc.VectorSubcoreMesh(
            core_axis_name="c", subcore_axis_name="s",
            num_cores=_NC, num_subcores=_NS,
        ),
        scratch_types=(
            [
                pltpu.VMEM((s_per_w, _N_LAST, _BLK), jnp.int32),
                pltpu.VMEM((_N_LAST, _BLK, D), jnp.float32),
                pltpu.VMEM((_BLK, D), jnp.float32),
                pltpu.VMEM((_BLK, D), jnp.float32),
                pltpu.VMEM((1, D), jnp.float32),
            ]
            + [pltpu.SemaphoreType.DMA] * _N_LAST
        ),
        compiler_params=pltpu.CompilerParams(use_tc_tiling_on_sc=False),
    )


def kernel(input, LI_emb, UI_emb, n_iter):
    B, L = input.shape
    V, D = LI_emb.shape

    last = lax.dynamic_slice_in_dim(input, L - n_iter, _N_LAST, axis=1)

    n_blocks = B // _BLK
    idx_arr = last.T.reshape(_N_LAST, n_blocks, _BLK).transpose(1, 0, 2)

    LI_pack = _make_tc_pack(V, D)(LI_emb.T)
    LI_lin = LI_pack.reshape(V, D)

    s_per_w = n_blocks // _NW
    item, user = _make_sc_call(B, D, n_blocks, s_per_w)(idx_arr, LI_lin, UI_emb)
    return item, user


# speedup vs baseline: 2.5648x; 1.8447x over previous
import functools

import jax
import jax.numpy as jnp
from jax import lax
from jax.experimental import pallas as pl
from jax.experimental.pallas import tpu as pltpu
from jax.experimental.pallas import tpu_sc as plsc

_NC = 2
_NS = 16
_NW = _NC * _NS
_LANES = 16

_N_LAST = 10
_BLK = 128

_TCHUNK = 7936
_NPAIR = 64
_OFF = _NPAIR * _TCHUNK


def _make_tc_pack(V, D):
    n_in_blocks = -(-V // _TCHUNK)

    def tbody(inA_ref, inB_ref, out_ref):
        cat = jnp.concatenate([inA_ref[...], inB_ref[...]], axis=0)
        out_ref[...] = cat.T

    return pl.pallas_call(
        tbody,
        grid=(_NPAIR,),
        in_specs=[
            pl.BlockSpec((D, _TCHUNK), lambda i: (0, i)),
            pl.BlockSpec(
                (D, _TCHUNK),
                lambda i: (0, jnp.minimum(i + _NPAIR, n_in_blocks - 1)),
            ),
        ],
        out_specs=pl.BlockSpec((_TCHUNK, 2 * D), lambda i: (i, 0)),
        out_shape=jax.ShapeDtypeStruct((_OFF, 2 * D), jnp.float32),
    )


def _make_sc_call(B, D, n_blocks, s_per_w):
    nq = D // _LANES

    def body(idx_hbm, li_hbm, ui_hbm, item_out, user_out,
             idx_v, bufs, acc, ub, ui_v, *sems):
        c = lax.axis_index("c")
        s = lax.axis_index("s")
        wid = s * _NC + c

        pltpu.sync_copy(idx_hbm.at[pl.ds(wid * s_per_w, s_per_w)], idx_v)
        pltpu.sync_copy(ui_hbm, ui_v)

        def fill_row(r, carry):
            for q in range(nq):
                sl = pl.ds(q * _LANES, _LANES)
                ub[r, sl] = ui_v[0, sl]
            return carry
        lax.fori_loop(0, _BLK, fill_row, 0, unroll=4)

        for t in range(s_per_w):
            row0 = (wid * s_per_w + t) * _BLK
            copies = [
                pltpu.async_copy(li_hbm.at[idx_v.at[t, j]], bufs.at[j], sems[j])
                for j in range(_N_LAST)
            ]
            for j in range(_N_LAST):
                copies[j].wait()

                def accum(r, carry, j=j):
                    for q in range(nq):
                        sl = pl.ds(q * _LANES, _LANES)
                        v = bufs[j, r, sl]
                        if j == 0:
                            acc[r, sl] = v
                        else:
                            acc[r, sl] += v
                    return carry
                lax.fori_loop(0, _BLK, accum, 0, unroll=8)

            pltpu.sync_copy(acc, item_out.at[pl.ds(row0, _BLK)])
            pltpu.sync_copy(ub, user_out.at[pl.ds(row0, _BLK)])

    return pl.kernel(
        body,
        out_type=(
            jax.ShapeDtypeStruct((B, D), jnp.float32),
            jax.ShapeDtypeStruct((B, D), jnp.float32),
        ),
        mesh=plsc.VectorSubcoreMesh(
            core_axis_name="c", subcore_axis_name="s",
            num_cores=_NC, num_subcores=_NS,
        ),
        scratch_types=(
            [
                pltpu.VMEM((s_per_w, _N_LAST, _BLK), jnp.int32),
                pltpu.VMEM((_N_LAST, _BLK, D), jnp.float32),
                pltpu.VMEM((_BLK, D), jnp.float32),
                pltpu.VMEM((_BLK, D), jnp.float32),
                pltpu.VMEM((1, D), jnp.float32),
            ]
            + [pltpu.SemaphoreType.DMA] * _N_LAST
        ),
        compiler_params=pltpu.CompilerParams(use_tc_tiling_on_sc=False),
    )


def kernel(input, LI_emb, UI_emb, n_iter):
    B, L = input.shape
    V, D = LI_emb.shape

    last = lax.dynamic_slice_in_dim(input, L - n_iter, _N_LAST, axis=1)

    last = jnp.where(last < _OFF, 2 * last, 2 * (last - _OFF) + 1)

    n_blocks = B // _BLK
    idx_arr = last.T.reshape(_N_LAST, n_blocks, _BLK).transpose(1, 0, 2)

    LI_pack = _make_tc_pack(V, D)(LI_emb.T, LI_emb.T)
    LI_lin = LI_pack.reshape(2 * _OFF, D)

    s_per_w = n_blocks // _NW
    item, user = _make_sc_call(B, D, n_blocks, s_per_w)(idx_arr, LI_lin, UI_emb)
    return item, user


# speedup vs baseline: 2.6906x; 1.0490x over previous
import functools

import jax
import jax.numpy as jnp
from jax import lax
from jax.experimental import pallas as pl
from jax.experimental.pallas import tpu as pltpu
from jax.experimental.pallas import tpu_sc as plsc

_NC = 2
_NS = 16
_NW = _NC * _NS
_LANES = 16

_N_LAST = 10
_BLK = 128

_TCHUNK = 7936
_NPAIR = 64
_OFF = _NPAIR * _TCHUNK


def _make_tc_pack(V, D):
    n_in_blocks = -(-V // _TCHUNK)

    def tbody(inA_ref, inB_ref, out_ref):
        cat = jnp.concatenate([inA_ref[...], inB_ref[...]], axis=0)
        out_ref[...] = cat.T

    return pl.pallas_call(
        tbody,
        grid=(_NPAIR,),
        in_specs=[
            pl.BlockSpec((D, _TCHUNK), lambda i: (0, i)),
            pl.BlockSpec(
                (D, _TCHUNK),
                lambda i: (0, jnp.minimum(i + _NPAIR, n_in_blocks - 1)),
            ),
        ],
        out_specs=pl.BlockSpec((_TCHUNK, 2 * D), lambda i: (i, 0)),
        out_shape=jax.ShapeDtypeStruct((_OFF, 2 * D), jnp.float32),
    )


def _make_sc_call(B, D, n_blocks, s_per_w):
    nq = D // _LANES

    def body(idx_hbm, li_hbm, item_out,
             idx_v, bufs, acc, *sems):
        c = lax.axis_index("c")
        s = lax.axis_index("s")
        wid = s * _NC + c

        pltpu.sync_copy(idx_hbm.at[pl.ds(wid * s_per_w, s_per_w)], idx_v)

        for t in range(s_per_w):
            row0 = (wid * s_per_w + t) * _BLK
            copies = [
                pltpu.async_copy(li_hbm.at[idx_v.at[t, j]], bufs.at[j], sems[j])
                for j in range(_N_LAST)
            ]
            for j in range(_N_LAST):
                copies[j].wait()

                def accum(r, carry, j=j):
                    for q in range(nq):
                        sl = pl.ds(q * _LANES, _LANES)
                        v = bufs[j, r, sl]
                        if j == 0:
                            acc[r, sl] = v
                        else:
                            acc[r, sl] += v
                    return carry
                lax.fori_loop(0, _BLK, accum, 0, unroll=8)

            pltpu.sync_copy(acc, item_out.at[pl.ds(row0, _BLK)])

    return pl.kernel(
        body,
        out_type=jax.ShapeDtypeStruct((B, D), jnp.float32),
        mesh=plsc.VectorSubcoreMesh(
            core_axis_name="c", subcore_axis_name="s",
            num_cores=_NC, num_subcores=_NS,
        ),
        scratch_types=(
            [
                pltpu.VMEM((s_per_w, _N_LAST, _BLK), jnp.int32),
                pltpu.VMEM((_N_LAST, _BLK, D), jnp.float32),
                pltpu.VMEM((_BLK, D), jnp.float32),
            ]
            + [pltpu.SemaphoreType.DMA] * _N_LAST
        ),
        compiler_params=pltpu.CompilerParams(use_tc_tiling_on_sc=False),
    )


def kernel(input, LI_emb, UI_emb, n_iter):
    B, L = input.shape
    V, D = LI_emb.shape

    last = lax.dynamic_slice_in_dim(input, L - n_iter, _N_LAST, axis=1)

    last = jnp.where(last < _OFF, 2 * last, 2 * (last - _OFF) + 1)

    n_blocks = B // _BLK
    idx_arr = last.T.reshape(_N_LAST, n_blocks, _BLK).transpose(1, 0, 2)

    LI_pack = _make_tc_pack(V, D)(LI_emb.T, LI_emb.T)
    LI_lin = LI_pack.reshape(2 * _OFF, D)

    s_per_w = n_blocks // _NW
    item = _make_sc_call(B, D, n_blocks, s_per_w)(idx_arr, LI_lin)
    user = jnp.broadcast_to(UI_emb, (B, UI_emb.shape[1]))
    return item, user
